# Initial kernel scaffold; baseline (speedup 1.0000x reference)
import jax
import jax.numpy as jnp
from jax.experimental import pallas as pl


def kernel(x, deep_table, wide_table, ln_gamma, ln_beta, W1, b1, W2, b2, Wp1, bp1, Wp2, bp2, Wk1, bk1, Wk2, bk2, Wl1, bl1, Wl2, bl2, Wwide):
    raise NotImplementedError("write your pallas kernel here")


# speedup vs baseline: 9.7969x; 9.7969x over previous
import functools

import jax
import jax.numpy as jnp
import numpy as np
from jax import lax
from jax.experimental import pallas as pl
from jax.experimental.pallas import tpu as pltpu
from jax.experimental.pallas import tpu_sc as plsc

D = 16
NC = 2
NS = 16
NW = NC * NS


def _make_gather(bf, c):
    per_w = bf // NW
    n_chunks = per_w // c
    mesh = plsc.VectorSubcoreMesh(core_axis_name="c", subcore_axis_name="s")

    @functools.partial(
        pl.kernel,
        mesh=mesh,
        compiler_params=pltpu.CompilerParams(use_tc_tiling_on_sc=False),
        out_type=[
            jax.ShapeDtypeStruct((bf, D), jnp.float32),
            jax.ShapeDtypeStruct((bf, D), jnp.float32),
        ],
        scratch_types=[
            pltpu.VMEM((c,), jnp.int32),
            pltpu.VMEM((c,), jnp.int32),
            pltpu.VMEM((c, D), jnp.float32),
            pltpu.VMEM((c, D), jnp.float32),
            pltpu.VMEM((c, D), jnp.float32),
            pltpu.VMEM((c, D), jnp.float32),
            pltpu.SemaphoreType.DMA,
            pltpu.SemaphoreType.DMA,
            pltpu.SemaphoreType.DMA,
            pltpu.SemaphoreType.DMA,
        ],
    )
    def gather_k(idx_hbm, deep_hbm, wide_hbm, deep_out, wide_out,
                 i0, i1, d0, d1, w0, w1, sd0, sd1, sw0, sw1):
        wid = lax.axis_index("s") * NC + lax.axis_index("c")
        base = wid * per_w
        ibufs = (i0, i1)
        dbufs = (d0, d1)
        wbufs = (w0, w1)
        sds = (sd0, sd1)
        sws = (sw0, sw1)
        handles = [None, None]

        def fire(g):
            b = g & 1
            off = base + g * c
            pltpu.sync_copy(idx_hbm.at[pl.ds(off, c)], ibufs[b])
            hd = pltpu.async_copy(deep_hbm.at[ibufs[b]], dbufs[b], sds[b])
            hw = pltpu.async_copy(wide_hbm.at[ibufs[b]], wbufs[b], sws[b])
            handles[b] = (hd, hw)

        fire(0)
        for g in range(n_chunks):
            if g + 1 < n_chunks:
                fire(g + 1)
            hd, hw = handles[g & 1]
            hd.wait()
            hw.wait()
            off = base + g * c
            pltpu.sync_copy(dbufs[g & 1], deep_out.at[pl.ds(off, c)])
            pltpu.sync_copy(wbufs[g & 1], wide_out.at[pl.ds(off, c)])

    return gather_k


def _softplus(x):
    return jnp.maximum(x, 0.0) + jnp.log1p(jnp.exp(-jnp.abs(x)))


def _dense_body(de, we, g_r, bt_r, w1_r, b1_r, w2_r, b2_r, wh1_r, bh1_r,
                wh2_r, bh2_r, ww_r, prop_o, k_o, l_o, wide_o):
    xb = de[...]
    mu = jnp.mean(xb, axis=-1, keepdims=True)
    var = jnp.mean((xb - mu) * (xb - mu), axis=-1, keepdims=True)
    h = (xb - mu) / jnp.sqrt(var + 1e-5) * g_r[...] + bt_r[...]
    h = jnp.maximum(jnp.dot(h, w1_r[...], preferred_element_type=jnp.float32)
                    + b1_r[...], 0.0)
    h = jnp.maximum(jnp.dot(h, w2_r[...], preferred_element_type=jnp.float32)
                    + b2_r[...], 0.0)
    h3 = jnp.maximum(jnp.dot(h, wh1_r[...], preferred_element_type=jnp.float32)
                     + bh1_r[...], 0.0)
    out3 = jnp.dot(h3, wh2_r[...], preferred_element_type=jnp.float32) + bh2_r[...]
    prop_o[...] = jax.nn.sigmoid(out3[:, 0:1])
    k_o[...] = jnp.maximum(_softplus(out3[:, 1:2]), 0.01)
    l_o[...] = jnp.maximum(_softplus(out3[:, 2:3]), 0.01)
    wide_o[...] = jnp.dot(we[...], ww_r[...], preferred_element_type=jnp.float32)


def _dense_call(deep_emb, wide_emb, ln_gamma, ln_beta, W1, b1, W2, b2,
                Wh1, bh1, Wh2, bh2, Wwide, bb):
    b = deep_emb.shape[0]
    inn = deep_emb.shape[1]
    grid = (b // bb,)

    def full(shape):
        return pl.BlockSpec(shape, lambda i: tuple(0 for _ in shape))

    return pl.pallas_call(
        _dense_body,
        grid=grid,
        in_specs=[
            pl.BlockSpec((bb, inn), lambda i: (i, 0)),
            pl.BlockSpec((bb, inn), lambda i: (i, 0)),
            full((inn,)), full((inn,)),
            full((inn, 512)), full((512,)),
            full((512, 256)), full((256,)),
            full((256, 384)), full((384,)),
            full((384, 3)), full((3,)),
            full((inn, 1)),
        ],
        out_specs=[
            pl.BlockSpec((bb, 1), lambda i: (i, 0)),
            pl.BlockSpec((bb, 1), lambda i: (i, 0)),
            pl.BlockSpec((bb, 1), lambda i: (i, 0)),
            pl.BlockSpec((bb, 1), lambda i: (i, 0)),
        ],
        out_shape=[jax.ShapeDtypeStruct((b, 1), jnp.float32)] * 4,
    )(deep_emb, wide_emb, ln_gamma, ln_beta, W1, b1, W2, b2,
      Wh1, bh1, Wh2, bh2, Wwide)


def kernel(x, deep_table, wide_table, ln_gamma, ln_beta, W1, b1, W2, b2,
           Wp1, bp1, Wp2, bp2, Wk1, bk1, Wk2, bk2, Wl1, bl1, Wl2, bl2, Wwide):
    b, f = x.shape
    bf = b * f
    idx = x.reshape(bf)

    gather = _make_gather(bf, 1664)
    deep_rows, wide_rows = gather(idx, deep_table, wide_table)
    deep_emb = deep_rows.reshape(b, f * D)
    wide_emb = wide_rows.reshape(b, f * D)

    Wh1 = jnp.concatenate([Wp1, Wk1, Wl1], axis=1)
    bh1 = jnp.concatenate([bp1, bk1, bl1], axis=0)
    z = jnp.zeros((128, 1), jnp.float32)
    Wh2 = jnp.concatenate([
        jnp.concatenate([Wp2, z, z], axis=1),
        jnp.concatenate([z, Wk2, z], axis=1),
        jnp.concatenate([z, z, Wl2], axis=1),
    ], axis=0)
    bh2 = jnp.concatenate([bp2, bk2, bl2], axis=0)

    prop, k_p, l_p, wide = _dense_call(
        deep_emb, wide_emb, ln_gamma, ln_beta, W1, b1, W2, b2,
        Wh1, bh1, Wh2, bh2, Wwide, 512)
    return (prop, k_p, l_p, wide)
